# Initial kernel scaffold; baseline (speedup 1.0000x reference)
import jax
import jax.numpy as jnp
from jax.experimental import pallas as pl


def kernel(x, edge_index, batch, W1_rel, b1, W1_root, W2_rel, b2, W2_root, Wc, bc):
    raise NotImplementedError("write your pallas kernel here")


# speedup vs baseline: 6.6757x; 6.6757x over previous
import functools

import jax
import jax.numpy as jnp
from jax import lax
from jax.experimental import pallas as pl
from jax.experimental.pallas import tpu as pltpu
from jax.experimental.pallas import tpu_sc as plsc

N = 10000
E = 320000
D = 128
H = 64
C = 10
G = 128

NC = 2
NS = 16
NW = NC * NS
CH = 128
NCH = 80
EP = NW * NCH * CH
NP = 10240
RPT = NP // NS

BN = 1000

def _segment_sum_body(table, src_idx, dst_idx, zeros, out, src_v, dst_v,
                      rows_v, acc, sem0, sem1):
    cid = lax.axis_index("c")
    sid = lax.axis_index("s")
    wid = cid * NS + sid

    pltpu.sync_copy(src_idx.at[wid], src_v)
    pltpu.sync_copy(dst_idx.at[wid], dst_v)
    pltpu.sync_copy(zeros.at[pl.ds(sid * RPT, RPT)],
                    acc.at[pl.ds(sid * RPT, RPT)])
    plsc.subcore_barrier()

    def gather(j, slot, sem):
        pltpu.make_async_copy(table.at[src_v.at[j]], rows_v.at[slot], sem).start()

    def gather_wait(j, slot, sem):
        pltpu.make_async_copy(table.at[src_v.at[j]], rows_v.at[slot], sem).wait()

    gather(0, 0, sem0)

    def body(jj, carry):
        j0 = jj * 2
        gather(j0 + 1, 1, sem1)
        gather_wait(j0, 0, sem0)
        pltpu.sync_copy(rows_v.at[0], acc.at[dst_v.at[j0]], add=True)

        @pl.when(jj + 1 < NCH // 2)
        def _():
            gather(j0 + 2, 0, sem0)

        gather_wait(j0 + 1, 1, sem1)
        pltpu.sync_copy(rows_v.at[1], acc.at[dst_v.at[j0 + 1]], add=True)
        return carry

    lax.fori_loop(0, NCH // 2, body, 0)
    plsc.subcore_barrier()
    pltpu.sync_copy(acc.at[pl.ds(sid * RPT, RPT)],
                    out.at[cid, pl.ds(sid * RPT, RPT)])


@functools.cache
def _segment_sum_sc():
    mesh = plsc.VectorSubcoreMesh(core_axis_name="c", subcore_axis_name="s")
    return pl.kernel(
        _segment_sum_body,
        mesh=mesh,
        out_type=jax.ShapeDtypeStruct((NC, NP, H), jnp.float32),
        scratch_types=[
            pltpu.VMEM((NCH, CH), jnp.int32),
            pltpu.VMEM((NCH, CH), jnp.int32),
            pltpu.VMEM((2, CH, H), jnp.float32),
            pltpu.VMEM_SHARED((NP, H), jnp.float32),
            pltpu.SemaphoreType.DMA,
            pltpu.SemaphoreType.DMA,
        ],
        compiler_params=pltpu.CompilerParams(use_tc_tiling_on_sc=False),
    )


def _proj2_body(x_ref, wa_ref, wb_ref, oa_ref, ob_ref):
    x = x_ref[...]
    dn = (((1,), (1,)), ((), ()))
    oa_ref[...] = lax.dot_general(x, wa_ref[...], dn,
                                  preferred_element_type=jnp.float32)
    ob_ref[...] = lax.dot_general(x, wb_ref[...], dn,
                                  preferred_element_type=jnp.float32)


def _proj2(x, wa, wb):
    d = x.shape[1]
    return pl.pallas_call(
        _proj2_body,
        grid=(N // BN,),
        in_specs=[
            pl.BlockSpec((BN, d), lambda i: (i, 0)),
            pl.BlockSpec((H, d), lambda i: (0, 0)),
            pl.BlockSpec((H, d), lambda i: (0, 0)),
        ],
        out_specs=[
            pl.BlockSpec((BN, H), lambda i: (i, 0)),
            pl.BlockSpec((BN, H), lambda i: (i, 0)),
        ],
        out_shape=[
            jax.ShapeDtypeStruct((N, H), jnp.float32),
            jax.ShapeDtypeStruct((N, H), jnp.float32),
        ],
    )(x, wa, wb)


def _layer_body(agg_ref, xr_ref, b_ref, wa_ref, wb_ref, oa_ref, ob_ref):
    h = jnp.maximum(agg_ref[0] + agg_ref[1] + xr_ref[...] + b_ref[...], 0.0)
    dn = (((1,), (1,)), ((), ()))
    oa_ref[...] = lax.dot_general(h, wa_ref[...], dn,
                                  preferred_element_type=jnp.float32)
    ob_ref[...] = lax.dot_general(h, wb_ref[...], dn,
                                  preferred_element_type=jnp.float32)


def _layer(agg, xr, b, wa, wb):
    return pl.pallas_call(
        _layer_body,
        grid=(N // BN,),
        in_specs=[
            pl.BlockSpec((NC, BN, H), lambda i: (0, i, 0)),
            pl.BlockSpec((BN, H), lambda i: (i, 0)),
            pl.BlockSpec((1, H), lambda i: (0, 0)),
            pl.BlockSpec((H, H), lambda i: (0, 0)),
            pl.BlockSpec((H, H), lambda i: (0, 0)),
        ],
        out_specs=[
            pl.BlockSpec((BN, H), lambda i: (i, 0)),
            pl.BlockSpec((BN, H), lambda i: (i, 0)),
        ],
        out_shape=[
            jax.ShapeDtypeStruct((N, H), jnp.float32),
            jax.ShapeDtypeStruct((N, H), jnp.float32),
        ],
    )(agg, xr, b, wa, wb)


def _pool_body(agg_ref, xr_ref, b_ref, batch_ref, wc_ref, bc_ref, out_ref,
               acc_ref):
    i = pl.program_id(0)

    @pl.when(i == 0)
    def _():
        acc_ref[...] = jnp.zeros_like(acc_ref)

    h = jnp.maximum(agg_ref[0] + agg_ref[1] + xr_ref[...] + b_ref[...], 0.0)
    ext = jnp.concatenate([h, jnp.ones((BN, 1), jnp.float32)], axis=1)
    onehot = (batch_ref[...] ==
              lax.broadcasted_iota(jnp.int32, (BN, G), 1)).astype(jnp.float32)
    acc_ref[...] += lax.dot_general(onehot, ext, (((0,), (0,)), ((), ())),
                                    preferred_element_type=jnp.float32)

    @pl.when(i == pl.num_programs(0) - 1)
    def _():
        sums = acc_ref[:, :H]
        cnt = acc_ref[:, H:H + 1]
        g = sums / jnp.maximum(cnt, 1.0)
        out_ref[...] = lax.dot_general(g, wc_ref[...], (((1,), (1,)), ((), ())),
                                       preferred_element_type=jnp.float32) \
            + bc_ref[...]


def _pool(agg, xr, b, batch2d, wc, bc2d):
    return pl.pallas_call(
        _pool_body,
        grid=(N // BN,),
        in_specs=[
            pl.BlockSpec((NC, BN, H), lambda i: (0, i, 0)),
            pl.BlockSpec((BN, H), lambda i: (i, 0)),
            pl.BlockSpec((1, H), lambda i: (0, 0)),
            pl.BlockSpec((BN, 1), lambda i: (i, 0)),
            pl.BlockSpec((C, H), lambda i: (0, 0)),
            pl.BlockSpec((1, C), lambda i: (0, 0)),
        ],
        out_specs=pl.BlockSpec((G, C), lambda i: (0, 0)),
        out_shape=jax.ShapeDtypeStruct((G, C), jnp.float32),
        scratch_shapes=[pltpu.VMEM((G, H + 1), jnp.float32)],
    )(agg, xr, b, batch2d, wc, bc2d)


def kernel(x, edge_index, batch, W1_rel, b1, W1_root, W2_rel, b2, W2_root,
           Wc, bc):
    src = edge_index[0]
    dst = edge_index[1]
    pad = EP - E
    srcp = jnp.concatenate([src, jnp.zeros((pad,), jnp.int32)]).reshape(
        NW, NCH, CH)
    dstp = jnp.concatenate([dst, jnp.full((pad,), N, jnp.int32)]).reshape(
        NW, NCH, CH)
    zeros = jnp.zeros((NP, H), jnp.float32)

    seg = _segment_sum_sc()
    xw1, xr1 = _proj2(x, W1_rel, W1_root)
    agg1 = seg(xw1, srcp, dstp, zeros)
    h1w2, h1r2 = _layer(agg1, xr1, b1.reshape(1, H), W2_rel, W2_root)
    agg2 = seg(h1w2, srcp, dstp, zeros)
    return _pool(agg2, h1r2, b2.reshape(1, H), batch.reshape(N, 1), Wc,
                 bc.reshape(1, C))
